# speedup vs baseline: 2.5478x; 2.5478x over previous
import functools

import jax
import jax.numpy as jnp
from jax.experimental import pallas as pl


def _mm_kernel(x_ref, w_ref, o_ref):
    o_ref[...] = jnp.dot(x_ref[...], w_ref[...],
                         preferred_element_type=jnp.float32)


def _matmul(x, w, block_rows=1000):
    n, k = x.shape
    _, m = w.shape
    grid = (n // block_rows,)
    return pl.pallas_call(
        _mm_kernel,
        grid=grid,
        in_specs=[
            pl.BlockSpec((block_rows, k), lambda i: (i, 0)),
            pl.BlockSpec((k, m), lambda i: (0, 0)),
        ],
        out_specs=pl.BlockSpec((block_rows, m), lambda i: (i, 0)),
        out_shape=jax.ShapeDtypeStruct((n, m), jnp.float32),
    )(x, w)


def _log_softmax_kernel(x_ref, o_ref):
    x = x_ref[...]
    m = jnp.max(x, axis=1, keepdims=True)
    s = x - m
    lse = jnp.log(jnp.sum(jnp.exp(s), axis=1, keepdims=True))
    o_ref[...] = s - lse


def _log_softmax(x, block_rows=1000):
    n, m = x.shape
    return pl.pallas_call(
        _log_softmax_kernel,
        grid=(n // block_rows,),
        in_specs=[pl.BlockSpec((block_rows, m), lambda i: (i, 0))],
        out_specs=pl.BlockSpec((block_rows, m), lambda i: (i, 0)),
        out_shape=jax.ShapeDtypeStruct((n, m), jnp.float32),
    )(x)


def kernel(x, edge_index, W1, b1, W2, b2):
    n = x.shape[0]
    src = edge_index[0]
    dst = edge_index[1]

    deg = jnp.zeros((n,), jnp.float32).at[dst].add(1.0) + 1.0
    d = deg ** -0.5
    dcol = d[:, None]

    h1 = _matmul(x, W1)
    y1 = h1 * dcol
    z1 = jnp.zeros_like(y1).at[dst].add(y1[src])
    out1 = jax.nn.relu(dcol * (z1 + y1) + b1)

    h2 = _matmul(out1, W2)
    y2 = h2 * dcol
    z2 = jnp.zeros_like(y2).at[dst].add(y2[src])
    out2 = dcol * (z2 + y2) + b2
    return _log_softmax(out2)


# speedup vs baseline: 7.7246x; 3.0318x over previous
import functools

import jax
import jax.numpy as jnp
from jax import lax
from jax.experimental import pallas as pl
from jax.experimental.pallas import tpu as pltpu
from jax.experimental.pallas import tpu_sc as plsc

_NC = 2
_NS = 16
_CK = 128


def _sc_mesh():
    return plsc.VectorSubcoreMesh(core_axis_name="c", subcore_axis_name="s")


def _make_deg(n_chunks, acc_rows):
    rpt = acc_rows // _NS
    cpt = n_chunks // (_NC * _NS)

    @functools.partial(
        pl.kernel,
        out_type=jax.ShapeDtypeStruct((_NC, acc_rows, 16), jnp.float32),
        mesh=_sc_mesh(),
        scratch_types=[
            pltpu.VMEM((cpt, _CK), jnp.int32),
            pltpu.VMEM((_CK, 16), jnp.float32),
            pltpu.VMEM_SHARED((acc_rows, 16), jnp.float32),
        ],
    )
    def deg_kernel(dst_hbm, zeros_hbm, out_hbm, idx, ones_v, acc):
        c = lax.axis_index("c")
        s = lax.axis_index("s")
        pltpu.sync_copy(zeros_hbm.at[pl.ds(s * rpt, rpt), :],
                        acc.at[pl.ds(s * rpt, rpt), :])

        @pl.loop(0, _CK)
        def _(j):
            ones_v.at[j][...] = jnp.full((16,), 1.0, jnp.float32)

        base = (c * _NS + s) * cpt
        pltpu.sync_copy(dst_hbm.at[pl.ds(base, cpt), :], idx)
        plsc.subcore_barrier()

        @pl.loop(0, cpt)
        def _(j):
            pltpu.sync_copy(ones_v, acc.at[idx.at[j]], add=True)

        plsc.subcore_barrier()
        pltpu.sync_copy(acc.at[pl.ds(s * rpt, rpt), :],
                        out_hbm.at[c, pl.ds(s * rpt, rpt), :])

    return deg_kernel


def _make_prop(n_chunks, acc_rows, feat, edge_split):
    rpt = acc_rows // _NS
    if edge_split:
        cpt = n_chunks // (_NC * _NS)
        phases = 1
    else:
        cpt = n_chunks // _NS
        phases = 2
    cpt_ph = cpt // phases

    @functools.partial(
        pl.kernel,
        out_type=jax.ShapeDtypeStruct((_NC, acc_rows, feat), jnp.float32),
        mesh=_sc_mesh(),
        scratch_types=[
            pltpu.VMEM((cpt_ph, _CK), jnp.int32),
            pltpu.VMEM((cpt_ph, _CK), jnp.int32),
            pltpu.VMEM((_CK, feat), jnp.float32),
            pltpu.VMEM((_CK, feat), jnp.float32),
            pltpu.VMEM_SHARED((acc_rows, feat), jnp.float32),
            pltpu.SemaphoreType.DMA,
            pltpu.SemaphoreType.DMA,
        ],
    )
    def prop_kernel(table_hbm, src_hbm, dst_hbm, zeros_hbm, out_hbm,
                    isrc, idst, rows0, rows1, acc, sem0, sem1):
        c = lax.axis_index("c")
        s = lax.axis_index("s")
        pltpu.sync_copy(zeros_hbm.at[pl.ds(s * rpt, rpt), :],
                        acc.at[pl.ds(s * rpt, rpt), :])
        plsc.subcore_barrier()

        for ph in range(phases):
            if edge_split:
                base = (c * _NS + s) * cpt + ph * cpt_ph
                pltpu.sync_copy(src_hbm.at[pl.ds(base, cpt_ph), :], isrc)
            else:
                base = s * cpt + ph * cpt_ph
                pltpu.sync_copy(src_hbm.at[c, pl.ds(base, cpt_ph), :], isrc)
            pltpu.sync_copy(dst_hbm.at[pl.ds(base, cpt_ph), :], idst)

            pltpu.async_copy(table_hbm.at[isrc.at[0]], rows0, sem0)

            @pl.loop(0, cpt_ph // 2)
            def _(j2):
                j = j2 * 2
                pltpu.async_copy(table_hbm.at[isrc.at[j + 1]], rows1, sem1)
                pltpu.make_async_copy(table_hbm.at[isrc.at[j]], rows0,
                                      sem0).wait()
                pltpu.sync_copy(rows0, acc.at[idst.at[j]], add=True)

                @pl.when(j2 < cpt_ph // 2 - 1)
                def _():
                    pltpu.async_copy(table_hbm.at[isrc.at[j + 2]], rows0, sem0)

                pltpu.make_async_copy(table_hbm.at[isrc.at[j + 1]], rows1,
                                      sem1).wait()
                pltpu.sync_copy(rows1, acc.at[idst.at[j + 1]], add=True)

        plsc.subcore_barrier()
        pltpu.sync_copy(acc.at[pl.ds(s * rpt, rpt), :],
                        out_hbm.at[c, pl.ds(s * rpt, rpt), :])

    return prop_kernel


def _mm_kernel(x_ref, w_ref, o_ref):
    o_ref[...] = jnp.dot(x_ref[...], w_ref[...],
                         preferred_element_type=jnp.float32)


def _matmul(x, w, block_rows=1000):
    n, k = x.shape
    _, m = w.shape
    return pl.pallas_call(
        _mm_kernel,
        grid=(n // block_rows,),
        in_specs=[
            pl.BlockSpec((block_rows, k), lambda i: (i, 0)),
            pl.BlockSpec((k, m), lambda i: (0, 0)),
        ],
        out_specs=pl.BlockSpec((block_rows, m), lambda i: (i, 0)),
        out_shape=jax.ShapeDtypeStruct((n, m), jnp.float32),
    )(x, w)


def _dcol(deg_ref):
    g = deg_ref[...]
    deg = g[0, :, 0] + g[1, :, 0] + 1.0
    return lax.rsqrt(deg)[:, None]


def _scale1_kernel(h_ref, deg_ref, o_ref):
    o_ref[0] = h_ref[...] * _dcol(deg_ref)


def _scale1(h1, deg2, block_rows=1000):
    n, m = h1.shape
    return pl.pallas_call(
        _scale1_kernel,
        grid=(n // block_rows, m // 128),
        in_specs=[
            pl.BlockSpec((block_rows, 128), lambda i, j: (i, j)),
            pl.BlockSpec((2, block_rows, 16), lambda i, j: (0, i, 0)),
        ],
        out_specs=pl.BlockSpec((1, block_rows, 128), lambda i, j: (j, i, 0)),
        out_shape=jax.ShapeDtypeStruct((m // 128, n, 128), jnp.float32),
    )(h1, deg2)


def _mid_kernel(z_ref, y_ref, deg_ref, b1_ref, w2_ref, o_ref):
    d = _dcol(deg_ref)
    zz = z_ref[...]
    yy = y_ref[...]
    t = jnp.concatenate([zz[0] + yy[0], zz[1] + yy[1]], axis=1)
    out1 = jax.nn.relu(t * d + b1_ref[...])
    o_ref[...] = jnp.dot(out1, w2_ref[...],
                         preferred_element_type=jnp.float32) * d


def _mid(z1, y1s, deg2, b1, W2, block_rows=1000):
    n = y1s.shape[1]
    m = W2.shape[1]
    return pl.pallas_call(
        _mid_kernel,
        grid=(n // block_rows,),
        in_specs=[
            pl.BlockSpec((2, block_rows, 128), lambda i: (0, i, 0)),
            pl.BlockSpec((2, block_rows, 128), lambda i: (0, i, 0)),
            pl.BlockSpec((2, block_rows, 16), lambda i: (0, i, 0)),
            pl.BlockSpec((1, 256), lambda i: (0, 0)),
            pl.BlockSpec((256, m), lambda i: (0, 0)),
        ],
        out_specs=pl.BlockSpec((block_rows, m), lambda i: (i, 0)),
        out_shape=jax.ShapeDtypeStruct((n, m), jnp.float32),
    )(z1, y1s, deg2, b1, W2)


def _final_kernel(z_ref, y_ref, deg_ref, b2_ref, o_ref):
    d = _dcol(deg_ref)
    zz = z_ref[...]
    m = y_ref.shape[1]
    t = (zz[0] + zz[1])[:, :m] + y_ref[...]
    v = t * d + b2_ref[...]
    m = jnp.max(v, axis=1, keepdims=True)
    sh = v - m
    lse = jnp.log(jnp.sum(jnp.exp(sh), axis=1, keepdims=True))
    o_ref[...] = sh - lse


def _final(z2, y2, deg2, b2, block_rows=1000):
    n, m = y2.shape
    return pl.pallas_call(
        _final_kernel,
        grid=(n // block_rows,),
        in_specs=[
            pl.BlockSpec((2, block_rows, 128), lambda i: (0, i, 0)),
            pl.BlockSpec((block_rows, m), lambda i: (i, 0)),
            pl.BlockSpec((2, block_rows, 16), lambda i: (0, i, 0)),
            pl.BlockSpec((1, m), lambda i: (0, 0)),
        ],
        out_specs=pl.BlockSpec((block_rows, m), lambda i: (i, 0)),
        out_shape=jax.ShapeDtypeStruct((n, m), jnp.float32),
    )(z2, y2, deg2, b2)


def kernel(x, edge_index, W1, b1, W2, b2):
    n = x.shape[0]
    e = edge_index.shape[1]
    src = edge_index[0]
    dst = edge_index[1]

    ept = _CK * 256
    e_pad = -(-e // ept) * ept
    n_chunks = e_pad // _CK
    acc_rows = -(-(n + 1) // (8 * _NS)) * (8 * _NS)
    trash = n

    pad = e_pad - e
    src2 = jnp.stack([src, src + n])
    src2 = jnp.concatenate(
        [src2, jnp.zeros((2, pad), jnp.int32)], axis=1)
    srcp = src2[0].reshape(n_chunks, _CK)
    src2 = src2.reshape(2, n_chunks, _CK)
    dstp = jnp.concatenate(
        [dst, jnp.full((pad,), trash, jnp.int32)])
    dstp = dstp.reshape(n_chunks, _CK)

    zeros16 = jnp.zeros((acc_rows, 16), jnp.float32)
    zeros128 = jnp.zeros((acc_rows, 128), jnp.float32)

    deg2 = _make_deg(n_chunks, acc_rows)(dstp, zeros16)

    h1 = _matmul(x, W1)
    y1s = _scale1(h1, deg2)

    z1 = _make_prop(n_chunks, acc_rows, 128, False)(
        y1s.reshape(2 * n, 128), src2, dstp, zeros128)

    y2 = _mid(z1, y1s, deg2, b1[None, :], W2)

    t2 = jnp.pad(y2, ((0, 0), (0, 64)))
    z2 = _make_prop(n_chunks, acc_rows, 128, True)(
        t2, srcp, dstp, zeros128)

    return _final(z2, y2, deg2, b2[None, :])


# speedup vs baseline: 18.9718x; 2.4560x over previous
import functools

import jax
import jax.numpy as jnp
from jax import lax
from jax.experimental import pallas as pl
from jax.experimental.pallas import tpu as pltpu
from jax.experimental.pallas import tpu_sc as plsc

_NC = 2
_NS = 16
_CK = 128


def _sc_mesh():
    return plsc.VectorSubcoreMesh(core_axis_name="c", subcore_axis_name="s")


def _make_deg(n_chunks, acc_rows):
    rpt = acc_rows // _NS
    cpt = n_chunks // (_NC * _NS)

    @functools.partial(
        pl.kernel,
        out_type=jax.ShapeDtypeStruct((_NC, acc_rows, 16), jnp.float32),
        mesh=_sc_mesh(),
        scratch_types=[
            pltpu.VMEM((cpt, _CK), jnp.int32),
            pltpu.VMEM((_CK, 16), jnp.float32),
            pltpu.VMEM_SHARED((acc_rows, 16), jnp.float32),
        ],
    )
    def deg_kernel(dst_hbm, zeros_hbm, out_hbm, idx, ones_v, acc):
        c = lax.axis_index("c")
        s = lax.axis_index("s")
        pltpu.sync_copy(zeros_hbm.at[pl.ds(s * rpt, rpt), :],
                        acc.at[pl.ds(s * rpt, rpt), :])

        @pl.loop(0, _CK)
        def _(j):
            ones_v.at[j][...] = jnp.full((16,), 1.0, jnp.float32)

        base = (c * _NS + s) * cpt
        pltpu.sync_copy(dst_hbm.at[pl.ds(base, cpt), :], idx)
        plsc.subcore_barrier()

        @pl.loop(0, cpt)
        def _(j):
            pltpu.sync_copy(ones_v, acc.at[idx.at[j]], add=True)

        plsc.subcore_barrier()
        pltpu.sync_copy(acc.at[pl.ds(s * rpt, rpt), :],
                        out_hbm.at[c, pl.ds(s * rpt, rpt), :])

    return deg_kernel


def _make_prop(n_chunks, acc_rows, feat, edge_split):
    rpt = acc_rows // _NS
    if edge_split:
        cpt = n_chunks // (_NC * _NS)
        phases = 1
    else:
        cpt = n_chunks // _NS
        phases = 2
    cpt_ph = cpt // phases

    @functools.partial(
        pl.kernel,
        out_type=jax.ShapeDtypeStruct((_NC, acc_rows, feat), jnp.float32),
        mesh=_sc_mesh(),
        scratch_types=[
            pltpu.VMEM((cpt_ph, _CK), jnp.int32),
            pltpu.VMEM((cpt_ph, _CK), jnp.int32),
            pltpu.VMEM((_CK, feat), jnp.float32),
            pltpu.VMEM((_CK, feat), jnp.float32),
            pltpu.VMEM_SHARED((acc_rows, feat), jnp.float32),
            pltpu.SemaphoreType.DMA,
            pltpu.SemaphoreType.DMA,
        ],
    )
    def prop_kernel(table_hbm, src_hbm, dst_hbm, zeros_hbm, out_hbm,
                    isrc, idst, rows0, rows1, acc, sem0, sem1):
        c = lax.axis_index("c")
        s = lax.axis_index("s")
        pltpu.sync_copy(zeros_hbm.at[pl.ds(s * rpt, rpt), :],
                        acc.at[pl.ds(s * rpt, rpt), :])
        plsc.subcore_barrier()

        for ph in range(phases):
            if edge_split:
                base = (c * _NS + s) * cpt + ph * cpt_ph
                pltpu.sync_copy(src_hbm.at[pl.ds(base, cpt_ph), :], isrc)
            else:
                base = s * cpt + ph * cpt_ph
                pltpu.sync_copy(src_hbm.at[c, pl.ds(base, cpt_ph), :], isrc)
            pltpu.sync_copy(dst_hbm.at[pl.ds(base, cpt_ph), :], idst)

            pltpu.async_copy(table_hbm.at[isrc.at[0]], rows0, sem0)

            @pl.loop(0, cpt_ph // 2)
            def _(j2):
                j = j2 * 2
                pltpu.async_copy(table_hbm.at[isrc.at[j + 1]], rows1, sem1)
                pltpu.make_async_copy(table_hbm.at[isrc.at[j]], rows0,
                                      sem0).wait()
                pltpu.sync_copy(rows0, acc.at[idst.at[j]], add=True)

                @pl.when(j2 < cpt_ph // 2 - 1)
                def _():
                    pltpu.async_copy(table_hbm.at[isrc.at[j + 2]], rows0, sem0)

                pltpu.make_async_copy(table_hbm.at[isrc.at[j + 1]], rows1,
                                      sem1).wait()
                pltpu.sync_copy(rows1, acc.at[idst.at[j + 1]], add=True)

        plsc.subcore_barrier()
        pltpu.sync_copy(acc.at[pl.ds(s * rpt, rpt), :],
                        out_hbm.at[c, pl.ds(s * rpt, rpt), :])

    return prop_kernel


def _mm_kernel(x_ref, w_ref, o_ref):
    o_ref[...] = jnp.dot(x_ref[...], w_ref[...],
                         preferred_element_type=jnp.float32)


def _matmul(x, w, block_rows=1000):
    n, k = x.shape
    _, m = w.shape
    return pl.pallas_call(
        _mm_kernel,
        grid=(n // block_rows,),
        in_specs=[
            pl.BlockSpec((block_rows, k), lambda i: (i, 0)),
            pl.BlockSpec((k, m), lambda i: (0, 0)),
        ],
        out_specs=pl.BlockSpec((block_rows, m), lambda i: (i, 0)),
        out_shape=jax.ShapeDtypeStruct((n, m), jnp.float32),
    )(x, w)


def _dcol(deg_ref):
    g = deg_ref[...]
    deg = g[0, :, 0] + g[1, :, 0] + 1.0
    return lax.rsqrt(deg)[:, None]


def _scale1_kernel(h_ref, deg_ref, o_ref):
    o_ref[0] = h_ref[...] * _dcol(deg_ref)


def _scale1(h1, deg2, block_rows=1000):
    n, m = h1.shape
    return pl.pallas_call(
        _scale1_kernel,
        grid=(n // block_rows, m // 128),
        in_specs=[
            pl.BlockSpec((block_rows, 128), lambda i, j: (i, j)),
            pl.BlockSpec((2, block_rows, 16), lambda i, j: (0, i, 0)),
        ],
        out_specs=pl.BlockSpec((1, block_rows, 128), lambda i, j: (j, i, 0)),
        out_shape=jax.ShapeDtypeStruct((m // 128, n, 128), jnp.float32),
    )(h1, deg2)


def _mid_kernel(z_ref, y_ref, deg_ref, b1_ref, w2_ref, o_ref):
    d = _dcol(deg_ref)
    zz = z_ref[...]
    yy = y_ref[...]
    t = jnp.concatenate([zz[0] + yy[0], zz[1] + yy[1]], axis=1)
    out1 = jax.nn.relu(t * d + b1_ref[...])
    o_ref[...] = jnp.dot(out1, w2_ref[...],
                         preferred_element_type=jnp.float32) * d


def _mid(z1, y1s, deg2, b1, W2, block_rows=1000):
    n = y1s.shape[1]
    m = W2.shape[1]
    return pl.pallas_call(
        _mid_kernel,
        grid=(n // block_rows,),
        in_specs=[
            pl.BlockSpec((2, block_rows, 128), lambda i: (0, i, 0)),
            pl.BlockSpec((2, block_rows, 128), lambda i: (0, i, 0)),
            pl.BlockSpec((2, block_rows, 16), lambda i: (0, i, 0)),
            pl.BlockSpec((1, 256), lambda i: (0, 0)),
            pl.BlockSpec((256, m), lambda i: (0, 0)),
        ],
        out_specs=pl.BlockSpec((block_rows, m), lambda i: (i, 0)),
        out_shape=jax.ShapeDtypeStruct((n, m), jnp.float32),
    )(z1, y1s, deg2, b1, W2)


def _final_kernel(z_ref, y_ref, deg_ref, b2_ref, o_ref):
    d = _dcol(deg_ref)
    zz = z_ref[...]
    m = y_ref.shape[1]
    t = (zz[0] + zz[1])[:, :m] + y_ref[...]
    v = t * d + b2_ref[...]
    m = jnp.max(v, axis=1, keepdims=True)
    sh = v - m
    lse = jnp.log(jnp.sum(jnp.exp(sh), axis=1, keepdims=True))
    o_ref[...] = sh - lse


def _final(z2, y2, deg2, b2, block_rows=1000):
    n, m = y2.shape
    return pl.pallas_call(
        _final_kernel,
        grid=(n // block_rows,),
        in_specs=[
            pl.BlockSpec((2, block_rows, 128), lambda i: (0, i, 0)),
            pl.BlockSpec((block_rows, m), lambda i: (i, 0)),
            pl.BlockSpec((2, block_rows, 16), lambda i: (0, i, 0)),
            pl.BlockSpec((1, m), lambda i: (0, 0)),
        ],
        out_specs=pl.BlockSpec((block_rows, m), lambda i: (i, 0)),
        out_shape=jax.ShapeDtypeStruct((n, m), jnp.float32),
    )(z2, y2, deg2, b2)


def kernel(x, edge_index, W1, b1, W2, b2):
    n = x.shape[0]
    e = edge_index.shape[1]
    src = edge_index[0]
    dst = edge_index[1]

    ept = _CK * 256
    e_pad = -(-e // ept) * ept
    n_chunks = e_pad // _CK
    acc_rows = -(-(n + 1) // (8 * _NS)) * (8 * _NS)
    trash = n

    pad = e_pad - e
    pad_src = jnp.arange(pad, dtype=jnp.int32) % n
    src2 = jnp.stack([src, src + n])
    src2 = jnp.concatenate(
        [src2, jnp.stack([pad_src, pad_src + n])], axis=1)
    srcp = src2[0].reshape(n_chunks, _CK)
    src2 = src2.reshape(2, n_chunks, _CK)
    spare = acc_rows - trash
    pad_dst = trash + jnp.arange(pad, dtype=jnp.int32) % spare
    dstp = jnp.concatenate([dst, pad_dst])
    dstp = dstp.reshape(n_chunks, _CK)

    zeros16 = jnp.zeros((acc_rows, 16), jnp.float32)
    zeros128 = jnp.zeros((acc_rows, 128), jnp.float32)

    deg2 = _make_deg(n_chunks, acc_rows)(dstp, zeros16)

    h1 = _matmul(x, W1)
    y1s = _scale1(h1, deg2)

    z1 = _make_prop(n_chunks, acc_rows, 128, False)(
        y1s.reshape(2 * n, 128), src2, dstp, zeros128)

    y2 = _mid(z1, y1s, deg2, b1[None, :], W2)

    t2 = jnp.pad(y2, ((0, 0), (0, 64)))
    z2 = _make_prop(n_chunks, acc_rows, 128, True)(
        t2, srcp, dstp, zeros128)

    return _final(z2, y2, deg2, b2[None, :])
